# baseline (speedup 1.0000x reference)
import functools

import jax
import jax.numpy as jnp
import numpy as np
from jax import lax
from jax.experimental import pallas as pl
from jax.experimental.pallas import tpu as pltpu
from jax.experimental.pallas import tpu_sc as plsc

N = 10000
E = 160000
D = 200
NE = 4
STEPS = 6
G = 64

NC = 2
NS = 16
DH = 128
DP = 2 * DH
EPT = E // NS
BB = 125
NB = EPT // BB
CH = 40
NCH = NB // CH
ACCN = 10240
RPT = ACCN // NS
RB = 80
NRB = RPT // RB

_F32 = jnp.float32


def _make_sc_scatter():
    mesh = plsc.VectorSubcoreMesh(
        core_axis_name="c", subcore_axis_name="s", num_cores=NC, num_subcores=NS
    )

    @functools.partial(
        pl.kernel,
        out_type=jax.ShapeDtypeStruct((NC, ACCN, DH), _F32),
        mesh=mesh,
        scratch_types=[
            pltpu.VMEM((CH, BB), jnp.int32),
            pltpu.VMEM((CH, BB), jnp.int32),
            pltpu.VMEM((BB, DH), _F32),
            pltpu.VMEM((BB, DH), _F32),
            pltpu.VMEM_SHARED((ACCN, DH), _F32),
            pltpu.SemaphoreType.DMA,
            pltpu.SemaphoreType.DMA,
            pltpu.SemaphoreType.DMA,
            pltpu.SemaphoreType.DMA,
        ],
    )
    def sc_scatter(ht_hbm, fidx_hbm, dst_hbm, zeros_hbm, out_hbm,
                   fidx_v, dst_v, buf0, buf1, acc_sh, sem0, sem1, ssem0, ssem1):
        c = lax.axis_index("c")
        s = lax.axis_index("s")

        pltpu.sync_copy(zeros_hbm, buf0.at[pl.ds(0, RB)])

        @pl.loop(0, NRB)
        def _zero(r):
            pltpu.sync_copy(buf0.at[pl.ds(0, RB)],
                            acc_sh.at[pl.ds(s * RPT + r * RB, RB)])

        plsc.subcore_barrier()

        @pl.loop(0, NCH)
        def _chunk(k):
            pltpu.sync_copy(fidx_hbm.at[c, s, pl.ds(k * CH, CH)], fidx_v)
            pltpu.sync_copy(dst_hbm.at[s, pl.ds(k * CH, CH)], dst_v)
            pltpu.async_copy(ht_hbm.at[fidx_v.at[0]], buf0, sem0)

            @pl.loop(0, CH // 2)
            def _body(j2):
                b0 = 2 * j2
                pltpu.make_async_copy(ht_hbm.at[fidx_v.at[b0]], buf0, sem0).wait()

                @pl.when(j2 > 0)
                def _():
                    pltpu.make_async_copy(
                        buf1, acc_sh.at[dst_v.at[b0 - 1]], ssem1).wait()

                pltpu.async_copy(ht_hbm.at[fidx_v.at[b0 + 1]], buf1, sem1)
                pltpu.async_copy(buf0, acc_sh.at[dst_v.at[b0]], ssem0, add=True)
                pltpu.make_async_copy(ht_hbm.at[fidx_v.at[b0 + 1]], buf1,
                                      sem1).wait()

                @pl.when(b0 + 2 < CH)
                def _():
                    pltpu.make_async_copy(
                        buf0, acc_sh.at[dst_v.at[b0]], ssem0).wait()
                    pltpu.async_copy(ht_hbm.at[fidx_v.at[b0 + 2]], buf0, sem0)

                pltpu.async_copy(buf1, acc_sh.at[dst_v.at[b0 + 1]], ssem1,
                                 add=True)

            pltpu.make_async_copy(buf0, acc_sh.at[dst_v.at[CH - 2]], ssem0).wait()
            pltpu.make_async_copy(buf1, acc_sh.at[dst_v.at[CH - 1]], ssem1).wait()

        plsc.subcore_barrier()

        @pl.loop(0, NRB)
        def _copyout(r):
            base = s * RPT + r * RB
            pltpu.sync_copy(acc_sh.at[pl.ds(base, RB)], buf0.at[pl.ds(0, RB)])
            pltpu.sync_copy(buf0.at[pl.ds(0, RB)], out_hbm.at[c, pl.ds(base, RB)])

    return sc_scatter


_sc_cache = []


def _sc_scatter(ht, fidx, dsti, zeros):
    if not _sc_cache:
        _sc_cache.append(_make_sc_scatter())
    return _sc_cache[0](ht, fidx, dsti, zeros)


_BN = 1000
_NBLK = N // _BN


def _ht_body(h_ref, wm_ref, bm_ref, ht_ref):
    h = h_ref[...]
    for t in range(NE):
        full = jnp.dot(h, wm_ref[t], preferred_element_type=_F32) + bm_ref[t]
        ht_ref[0, t] = full[:, 0:DH]
        ht_ref[1, t] = full[:, DH:DP]


def _ht_kernel(h, wmT, bm):
    return pl.pallas_call(
        _ht_body,
        grid=(_NBLK,),
        in_specs=[
            pl.BlockSpec((_BN, D), lambda i: (i, 0)),
            pl.BlockSpec((NE, D, DP), lambda i: (0, 0, 0)),
            pl.BlockSpec((NE, 1, DP), lambda i: (0, 0, 0)),
        ],
        out_specs=pl.BlockSpec((NC, NE, _BN, DH), lambda i: (0, 0, i, 0)),
        out_shape=jax.ShapeDtypeStruct((NC, NE, N, DH), _F32),
    )(h, wmT, bm)


def _gh_body(h_ref, whh_ref, bhh_ref, gh_ref):
    gh_ref[...] = (jnp.dot(h_ref[...], whh_ref[...],
                           preferred_element_type=_F32) + bhh_ref[...])


def _gh_kernel(h, whhT, bhh):
    return pl.pallas_call(
        _gh_body,
        grid=(_NBLK,),
        in_specs=[
            pl.BlockSpec((_BN, D), lambda i: (i, 0)),
            pl.BlockSpec((D, 3 * D), lambda i: (0, 0)),
            pl.BlockSpec((1, 3 * D), lambda i: (0, 0)),
        ],
        out_specs=pl.BlockSpec((_BN, 3 * D), lambda i: (i, 0)),
        out_shape=jax.ShapeDtypeStruct((N, 3 * D), _F32),
    )(h, whhT, bhh)


def _gru_body(h_ref, ap_ref, gh_ref, wih_ref, bih_ref,
              wm_ref, bm_ref, hnew_ref, ht_ref):
    h = h_ref[...]
    a = jnp.concatenate([ap_ref[0], ap_ref[1]], axis=1)[:, 0:D]
    gi = jnp.dot(a, wih_ref[...], preferred_element_type=_F32) + bih_ref[...]
    gh = gh_ref[...]
    i_r = gi[:, 0:D]
    i_z = gi[:, D:2 * D]
    i_n = gi[:, 2 * D:3 * D]
    h_r = gh[:, 0:D]
    h_z = gh[:, D:2 * D]
    h_n = gh[:, 2 * D:3 * D]
    r = jax.nn.sigmoid(i_r + h_r)
    z = jax.nn.sigmoid(i_z + h_z)
    n = jnp.tanh(i_n + r * h_n)
    hn = (1.0 - z) * n + z * h
    hnew_ref[...] = hn
    for t in range(NE):
        full = jnp.dot(hn, wm_ref[t], preferred_element_type=_F32) + bm_ref[t]
        ht_ref[0, t] = full[:, 0:DH]
        ht_ref[1, t] = full[:, DH:DP]


def _gru_kernel(h, ap, gh, wihT, bih, wmT, bm):
    return pl.pallas_call(
        _gru_body,
        grid=(_NBLK,),
        in_specs=[
            pl.BlockSpec((_BN, D), lambda i: (i, 0)),
            pl.BlockSpec((NC, _BN, DH), lambda i: (0, i, 0)),
            pl.BlockSpec((_BN, 3 * D), lambda i: (i, 0)),
            pl.BlockSpec((D, 3 * D), lambda i: (0, 0)),
            pl.BlockSpec((1, 3 * D), lambda i: (0, 0)),
            pl.BlockSpec((NE, D, DP), lambda i: (0, 0, 0)),
            pl.BlockSpec((NE, 1, DP), lambda i: (0, 0, 0)),
        ],
        out_specs=[
            pl.BlockSpec((_BN, D), lambda i: (i, 0)),
            pl.BlockSpec((NC, NE, _BN, DH), lambda i: (0, 0, i, 0)),
        ],
        out_shape=[
            jax.ShapeDtypeStruct((N, D), _F32),
            jax.ShapeDtypeStruct((NC, NE, N, DH), _F32),
        ],
    )(h, ap, gh, wihT, bih, wmT, bm)


def _gru_last_body(h_ref, ap_ref, gh_ref, wih_ref, bih_ref, hnew_ref):
    h = h_ref[...]
    a = jnp.concatenate([ap_ref[0], ap_ref[1]], axis=1)[:, 0:D]
    gi = jnp.dot(a, wih_ref[...], preferred_element_type=_F32) + bih_ref[...]
    gh = gh_ref[...]
    r = jax.nn.sigmoid(gi[:, 0:D] + gh[:, 0:D])
    z = jax.nn.sigmoid(gi[:, D:2 * D] + gh[:, D:2 * D])
    n = jnp.tanh(gi[:, 2 * D:3 * D] + r * gh[:, 2 * D:3 * D])
    hnew_ref[...] = (1.0 - z) * n + z * h


def _gru_last_kernel(h, ap, gh, wihT, bih):
    return pl.pallas_call(
        _gru_last_body,
        grid=(_NBLK,),
        in_specs=[
            pl.BlockSpec((_BN, D), lambda i: (i, 0)),
            pl.BlockSpec((NC, _BN, DH), lambda i: (0, i, 0)),
            pl.BlockSpec((_BN, 3 * D), lambda i: (i, 0)),
            pl.BlockSpec((D, 3 * D), lambda i: (0, 0)),
            pl.BlockSpec((1, 3 * D), lambda i: (0, 0)),
        ],
        out_specs=pl.BlockSpec((_BN, D), lambda i: (i, 0)),
        out_shape=jax.ShapeDtypeStruct((N, D), _F32),
    )(h, ap, gh, wihT, bih)


def _head_body(loc_ref, h_ref, gid_ref, y_ref, scal_ref,
               s3g_ref, s2g_ref, s3l_ref, s2l_ref, l1_ref, l2_ref, lb_ref,
               val_ref, pred_ref, loss_ref, seg_acc, cnt_acc):
    i = pl.program_id(0)
    w0 = scal_ref[0]
    w1 = scal_ref[1]
    w2 = scal_ref[2]
    cb1 = scal_ref[3]
    cw2 = scal_ref[4]
    cb2 = scal_ref[5]

    loc = loc_ref[...]
    x = jnp.concatenate([loc, h_ref[...]], axis=1)

    def conv_pool(v, s3, s2, n3):
        L = v.shape[1]
        y = jax.nn.relu(w0 * v[:, 0:L - 2] + w1 * v[:, 1:L - 1]
                        + w2 * v[:, 2:L] + cb1)
        p = jnp.dot(y, s3[0], preferred_element_type=_F32)
        for k in range(1, n3):
            p = jnp.maximum(p, jnp.dot(y, s3[k], preferred_element_type=_F32))
        q = jax.nn.relu(cw2 * p + cb2)
        r = jnp.maximum(
            jnp.dot(q, s2[0], preferred_element_type=_F32),
            jnp.dot(q, s2[1], preferred_element_type=_F32),
        )
        return r

    f_g = conv_pool(x, s3g_ref, s2g_ref, 3)
    f_l = conv_pool(loc, s3l_ref, s2l_ref, 3)
    zv = jnp.dot(f_g, l1_ref[...], preferred_element_type=_F32) + lb_ref[0:1, :]
    yv = jnp.dot(f_l, l2_ref[...], preferred_element_type=_F32) + lb_ref[1:2, :]
    res = zv * yv

    gid = gid_ref[...]
    gids = lax.broadcasted_iota(jnp.int32, (_BN, G), 1)
    oh = (gid == gids).astype(_F32)
    seg = lax.dot_general(oh, res, (((0,), (0,)), ((), ())),
                          preferred_element_type=_F32)
    cnt = lax.dot_general(oh, jnp.ones((_BN, 1), _F32),
                          (((0,), (0,)), ((), ())),
                          preferred_element_type=_F32)

    @pl.when(i == 0)
    def _():
        seg_acc[...] = jnp.zeros_like(seg_acc)
        cnt_acc[...] = jnp.zeros_like(cnt_acc)

    seg_acc[...] += seg
    cnt_acc[...] += cnt

    @pl.when(i == _NBLK - 1)
    def _():
        res_g = seg_acc[...] / jnp.maximum(cnt_acc[...], 1.0)
        m = jnp.max(res_g, axis=1, keepdims=True)
        ex = jnp.exp(res_g - m)
        se = jnp.sum(ex, axis=1, keepdims=True)
        probs = ex / se
        val_ref[...] = jnp.max(probs, axis=1, keepdims=True)
        p1gt = probs[:, 1:2] > probs[:, 0:1]
        pred_ref[...] = p1gt.astype(jnp.int32)
        logp = res_g - m - jnp.log(se)
        yv_ = y_ref[...]
        pick = jnp.where(yv_ == 1, logp[:, 1:2], logp[:, 0:1])
        loss_ref[...] = -jnp.sum(pick, keepdims=True).reshape(1, 1) / G


def _head_kernel(loc, h, gid2, y2, scal, s3g, s2g, s3l, s2l, l1T, l2T, lb):
    return pl.pallas_call(
        _head_body,
        grid=(_NBLK,),
        in_specs=[
            pl.BlockSpec((_BN, D), lambda i: (i, 0)),
            pl.BlockSpec((_BN, D), lambda i: (i, 0)),
            pl.BlockSpec((_BN, 1), lambda i: (i, 0)),
            pl.BlockSpec((G, 1), lambda i: (0, 0)),
            pl.BlockSpec(memory_space=pltpu.SMEM),
            pl.BlockSpec((3, 398, 198), lambda i: (0, 0, 0)),
            pl.BlockSpec((2, 198, 99), lambda i: (0, 0, 0)),
            pl.BlockSpec((3, 198, 98), lambda i: (0, 0, 0)),
            pl.BlockSpec((2, 98, 49), lambda i: (0, 0, 0)),
            pl.BlockSpec((99, 2), lambda i: (0, 0)),
            pl.BlockSpec((49, 2), lambda i: (0, 0)),
            pl.BlockSpec((2, 2), lambda i: (0, 0)),
        ],
        out_specs=[
            pl.BlockSpec((G, 1), lambda i: (0, 0)),
            pl.BlockSpec((G, 1), lambda i: (0, 0)),
            pl.BlockSpec((1, 1), lambda i: (0, 0)),
        ],
        out_shape=[
            jax.ShapeDtypeStruct((G, 1), _F32),
            jax.ShapeDtypeStruct((G, 1), jnp.int32),
            jax.ShapeDtypeStruct((1, 1), _F32),
        ],
        scratch_shapes=[
            pltpu.VMEM((G, 2), _F32),
            pltpu.VMEM((G, 1), _F32),
        ],
    )(loc, h, gid2, y2, scal, s3g, s2g, s3l, s2l, l1T, l2T, lb)


def _pool_selectors(L, k, s):
    P = (L - k) // s + 1
    S = np.zeros((k, L, P), np.float32)
    for j in range(P):
        for t in range(k):
            S[t, s * j + t, j] = 1.0
    return S


def kernel(local, edge_index, e_type, graph_ids, y, W_msg, b_msg, W_ih, W_hh,
           b_ih, b_hh, conv1_w, conv1_b, conv2_w, conv2_b, lin1_w, lin1_b,
           lin2_w, lin2_b):
    src = edge_index[0]
    dst = edge_index[1]

    fidx0 = (e_type.astype(jnp.int32) * N + src.astype(jnp.int32)).reshape(NS, NB, BB)
    fidx = jnp.stack([fidx0, fidx0 + NE * N])
    dsti = dst.astype(jnp.int32).reshape(NS, NB, BB)
    zeros = jnp.zeros((RB, DH), _F32)

    wmT = jnp.pad(W_msg.transpose(0, 2, 1),
                  ((0, 0), (0, 0), (0, DP - D)))
    bm = jnp.pad(b_msg, ((0, 0), (0, DP - D))).reshape(NE, 1, DP)
    wihT = W_ih.T
    whhT = W_hh.T
    bih = b_ih.reshape(1, 3 * D)
    bhh = b_hh.reshape(1, 3 * D)

    scal = jnp.stack([conv1_w[0, 0, 0], conv1_w[0, 0, 1], conv1_w[0, 0, 2],
                      conv1_b[0], conv2_w[0, 0, 0], conv2_b[0]])
    s3g = jnp.asarray(_pool_selectors(398, 3, 2))
    s2g = jnp.asarray(_pool_selectors(198, 2, 2))
    s3l = jnp.asarray(_pool_selectors(198, 3, 2))
    s2l = jnp.asarray(_pool_selectors(98, 2, 2))
    l1T = lin1_w.T
    l2T = lin2_w.T
    lb = jnp.stack([lin1_b, lin2_b])

    gid2 = graph_ids.astype(jnp.int32).reshape(N, 1)
    y2 = y.astype(jnp.int32).reshape(G, 1)

    h = local
    ht = _ht_kernel(h, wmT, bm).reshape(NC * NE * N, DH)
    for step in range(STEPS):
        ap = _sc_scatter(ht, fidx, dsti, zeros)
        gh = _gh_kernel(h, whhT, bhh)
        if step < STEPS - 1:
            h, ht4 = _gru_kernel(h, ap, gh, wihT, bih, wmT, bm)
            ht = ht4.reshape(NC * NE * N, DH)
        else:
            h = _gru_last_kernel(h, ap, gh, wihT, bih)

    val, pred, loss = _head_kernel(local, h, gid2, y2, scal, s3g, s2g, s3l,
                                   s2l, l1T, l2T, lb)
    return (val.reshape(G), pred.reshape(G), loss.reshape(()))


# speedup vs baseline: 1.0593x; 1.0593x over previous
import functools

import jax
import jax.numpy as jnp
import numpy as np
from jax import lax
from jax.experimental import pallas as pl
from jax.experimental.pallas import tpu as pltpu
from jax.experimental.pallas import tpu_sc as plsc

N = 10000
E = 160000
D = 200
NE = 4
STEPS = 6
G = 64

NC = 2
NS = 16
DH = 128
DP = 2 * DH
EPT = E // NS
BB = 125
NB = EPT // BB
CH = 40
NCH = NB // CH
ACCN = 10240
RPT = ACCN // NS
RB = 80
NRB = RPT // RB

_F32 = jnp.float32


def _make_sc_scatter():
    mesh = plsc.VectorSubcoreMesh(
        core_axis_name="c", subcore_axis_name="s", num_cores=NC, num_subcores=NS
    )

    @functools.partial(
        pl.kernel,
        out_type=jax.ShapeDtypeStruct((NC, ACCN, DH), _F32),
        mesh=mesh,
        scratch_types=[
            pltpu.VMEM((CH, BB), jnp.int32),
            pltpu.VMEM((CH, BB), jnp.int32),
            pltpu.VMEM((BB, DH), _F32),
            pltpu.VMEM((BB, DH), _F32),
            pltpu.VMEM_SHARED((ACCN, DH), _F32),
            pltpu.SemaphoreType.DMA,
            pltpu.SemaphoreType.DMA,
            pltpu.SemaphoreType.DMA,
            pltpu.SemaphoreType.DMA,
        ],
    )
    def sc_scatter(ht_hbm, fidx_hbm, dst_hbm, zeros_hbm, out_hbm,
                   fidx_v, dst_v, buf0, buf1, acc_sh, sem0, sem1, ssem0, ssem1):
        c = lax.axis_index("c")
        s = lax.axis_index("s")

        pltpu.sync_copy(zeros_hbm, buf0.at[pl.ds(0, RB)])

        @pl.loop(0, NRB)
        def _zero(r):
            pltpu.sync_copy(buf0.at[pl.ds(0, RB)],
                            acc_sh.at[pl.ds(s * RPT + r * RB, RB)])

        plsc.subcore_barrier()

        @pl.loop(0, NCH)
        def _chunk(k):
            pltpu.sync_copy(fidx_hbm.at[c, s, pl.ds(k * CH, CH)], fidx_v)
            pltpu.sync_copy(dst_hbm.at[s, pl.ds(k * CH, CH)], dst_v)
            pltpu.async_copy(ht_hbm.at[fidx_v.at[0]], buf0, sem0)

            @pl.loop(0, CH // 2)
            def _body(j2):
                b0 = 2 * j2
                pltpu.make_async_copy(ht_hbm.at[fidx_v.at[b0]], buf0, sem0).wait()

                @pl.when(j2 > 0)
                def _():
                    pltpu.make_async_copy(
                        buf1, acc_sh.at[dst_v.at[b0 - 1]], ssem1).wait()

                pltpu.async_copy(ht_hbm.at[fidx_v.at[b0 + 1]], buf1, sem1)
                pltpu.async_copy(buf0, acc_sh.at[dst_v.at[b0]], ssem0, add=True)
                pltpu.make_async_copy(ht_hbm.at[fidx_v.at[b0 + 1]], buf1,
                                      sem1).wait()

                @pl.when(b0 + 2 < CH)
                def _():
                    pltpu.make_async_copy(
                        buf0, acc_sh.at[dst_v.at[b0]], ssem0).wait()
                    pltpu.async_copy(ht_hbm.at[fidx_v.at[b0 + 2]], buf0, sem0)

                pltpu.async_copy(buf1, acc_sh.at[dst_v.at[b0 + 1]], ssem1,
                                 add=True)

            pltpu.make_async_copy(buf0, acc_sh.at[dst_v.at[CH - 2]], ssem0).wait()
            pltpu.make_async_copy(buf1, acc_sh.at[dst_v.at[CH - 1]], ssem1).wait()

        plsc.subcore_barrier()

        @pl.loop(0, NRB)
        def _copyout(r):
            base = s * RPT + r * RB
            pltpu.sync_copy(acc_sh.at[pl.ds(base, RB)], buf0.at[pl.ds(0, RB)])
            pltpu.sync_copy(buf0.at[pl.ds(0, RB)], out_hbm.at[c, pl.ds(base, RB)])

    return sc_scatter


_sc_cache = []


def _sc_scatter(ht, fidx, dsti, zeros):
    if not _sc_cache:
        _sc_cache.append(_make_sc_scatter())
    return _sc_cache[0](ht, fidx, dsti, zeros)


_BN = 1000
_NBLK = N // _BN


def _ht_body(h_ref, wm_ref, bm_ref, ht_ref):
    h = h_ref[...]
    for t in range(NE):
        full = jnp.dot(h, wm_ref[t], preferred_element_type=_F32) + bm_ref[t]
        ht_ref[0, t] = full[:, 0:DH]
        ht_ref[1, t] = full[:, DH:DP]


def _ht_kernel(h, wmT, bm):
    return pl.pallas_call(
        _ht_body,
        grid=(_NBLK,),
        in_specs=[
            pl.BlockSpec((_BN, D), lambda i: (i, 0)),
            pl.BlockSpec((NE, D, DP), lambda i: (0, 0, 0)),
            pl.BlockSpec((NE, 1, DP), lambda i: (0, 0, 0)),
        ],
        out_specs=pl.BlockSpec((NC, NE, _BN, DH), lambda i: (0, 0, i, 0)),
        out_shape=jax.ShapeDtypeStruct((NC, NE, N, DH), _F32),
    )(h, wmT, bm)


def _gru_body(h_ref, ap_ref, wih_ref, whh_ref, bih_ref, bhh_ref,
              wm_ref, bm_ref, hnew_ref, ht_ref):
    h = h_ref[...]
    a = jnp.concatenate([ap_ref[0], ap_ref[1]], axis=1)[:, 0:D]
    gi = jnp.dot(a, wih_ref[...], preferred_element_type=_F32) + bih_ref[...]
    gh = jnp.dot(h, whh_ref[...], preferred_element_type=_F32) + bhh_ref[...]
    i_r = gi[:, 0:D]
    i_z = gi[:, D:2 * D]
    i_n = gi[:, 2 * D:3 * D]
    h_r = gh[:, 0:D]
    h_z = gh[:, D:2 * D]
    h_n = gh[:, 2 * D:3 * D]
    r = jax.nn.sigmoid(i_r + h_r)
    z = jax.nn.sigmoid(i_z + h_z)
    n = jnp.tanh(i_n + r * h_n)
    hn = (1.0 - z) * n + z * h
    hnew_ref[...] = hn
    for t in range(NE):
        full = jnp.dot(hn, wm_ref[t], preferred_element_type=_F32) + bm_ref[t]
        ht_ref[0, t] = full[:, 0:DH]
        ht_ref[1, t] = full[:, DH:DP]


def _gru_kernel(h, ap, wihT, whhT, bih, bhh, wmT, bm):
    return pl.pallas_call(
        _gru_body,
        grid=(_NBLK,),
        in_specs=[
            pl.BlockSpec((_BN, D), lambda i: (i, 0)),
            pl.BlockSpec((NC, _BN, DH), lambda i: (0, i, 0)),
            pl.BlockSpec((D, 3 * D), lambda i: (0, 0)),
            pl.BlockSpec((D, 3 * D), lambda i: (0, 0)),
            pl.BlockSpec((1, 3 * D), lambda i: (0, 0)),
            pl.BlockSpec((1, 3 * D), lambda i: (0, 0)),
            pl.BlockSpec((NE, D, DP), lambda i: (0, 0, 0)),
            pl.BlockSpec((NE, 1, DP), lambda i: (0, 0, 0)),
        ],
        out_specs=[
            pl.BlockSpec((_BN, D), lambda i: (i, 0)),
            pl.BlockSpec((NC, NE, _BN, DH), lambda i: (0, 0, i, 0)),
        ],
        out_shape=[
            jax.ShapeDtypeStruct((N, D), _F32),
            jax.ShapeDtypeStruct((NC, NE, N, DH), _F32),
        ],
    )(h, ap, wihT, whhT, bih, bhh, wmT, bm)


def _gru_last_body(h_ref, ap_ref, wih_ref, whh_ref, bih_ref, bhh_ref,
                   hnew_ref):
    h = h_ref[...]
    a = jnp.concatenate([ap_ref[0], ap_ref[1]], axis=1)[:, 0:D]
    gi = jnp.dot(a, wih_ref[...], preferred_element_type=_F32) + bih_ref[...]
    gh = jnp.dot(h, whh_ref[...], preferred_element_type=_F32) + bhh_ref[...]
    r = jax.nn.sigmoid(gi[:, 0:D] + gh[:, 0:D])
    z = jax.nn.sigmoid(gi[:, D:2 * D] + gh[:, D:2 * D])
    n = jnp.tanh(gi[:, 2 * D:3 * D] + r * gh[:, 2 * D:3 * D])
    hnew_ref[...] = (1.0 - z) * n + z * h


def _gru_last_kernel(h, ap, wihT, whhT, bih, bhh):
    return pl.pallas_call(
        _gru_last_body,
        grid=(_NBLK,),
        in_specs=[
            pl.BlockSpec((_BN, D), lambda i: (i, 0)),
            pl.BlockSpec((NC, _BN, DH), lambda i: (0, i, 0)),
            pl.BlockSpec((D, 3 * D), lambda i: (0, 0)),
            pl.BlockSpec((D, 3 * D), lambda i: (0, 0)),
            pl.BlockSpec((1, 3 * D), lambda i: (0, 0)),
            pl.BlockSpec((1, 3 * D), lambda i: (0, 0)),
        ],
        out_specs=pl.BlockSpec((_BN, D), lambda i: (i, 0)),
        out_shape=jax.ShapeDtypeStruct((N, D), _F32),
    )(h, ap, wihT, whhT, bih, bhh)


def _head_body(loc_ref, h_ref, gid_ref, y_ref, scal_ref,
               s3g_ref, s2g_ref, s3l_ref, s2l_ref, l1_ref, l2_ref, lb_ref,
               val_ref, pred_ref, loss_ref, seg_acc, cnt_acc):
    i = pl.program_id(0)
    w0 = scal_ref[0]
    w1 = scal_ref[1]
    w2 = scal_ref[2]
    cb1 = scal_ref[3]
    cw2 = scal_ref[4]
    cb2 = scal_ref[5]

    loc = loc_ref[...]
    x = jnp.concatenate([loc, h_ref[...]], axis=1)

    def conv_pool(v, s3, s2, n3):
        L = v.shape[1]
        y = jax.nn.relu(w0 * v[:, 0:L - 2] + w1 * v[:, 1:L - 1]
                        + w2 * v[:, 2:L] + cb1)
        p = jnp.dot(y, s3[0], preferred_element_type=_F32)
        for k in range(1, n3):
            p = jnp.maximum(p, jnp.dot(y, s3[k], preferred_element_type=_F32))
        q = jax.nn.relu(cw2 * p + cb2)
        r = jnp.maximum(
            jnp.dot(q, s2[0], preferred_element_type=_F32),
            jnp.dot(q, s2[1], preferred_element_type=_F32),
        )
        return r

    f_g = conv_pool(x, s3g_ref, s2g_ref, 3)
    f_l = conv_pool(loc, s3l_ref, s2l_ref, 3)
    zv = jnp.dot(f_g, l1_ref[...], preferred_element_type=_F32) + lb_ref[0:1, :]
    yv = jnp.dot(f_l, l2_ref[...], preferred_element_type=_F32) + lb_ref[1:2, :]
    res = zv * yv

    gid = gid_ref[...]
    gids = lax.broadcasted_iota(jnp.int32, (_BN, G), 1)
    oh = (gid == gids).astype(_F32)
    seg = lax.dot_general(oh, res, (((0,), (0,)), ((), ())),
                          preferred_element_type=_F32)
    cnt = lax.dot_general(oh, jnp.ones((_BN, 1), _F32),
                          (((0,), (0,)), ((), ())),
                          preferred_element_type=_F32)

    @pl.when(i == 0)
    def _():
        seg_acc[...] = jnp.zeros_like(seg_acc)
        cnt_acc[...] = jnp.zeros_like(cnt_acc)

    seg_acc[...] += seg
    cnt_acc[...] += cnt

    @pl.when(i == _NBLK - 1)
    def _():
        res_g = seg_acc[...] / jnp.maximum(cnt_acc[...], 1.0)
        m = jnp.max(res_g, axis=1, keepdims=True)
        ex = jnp.exp(res_g - m)
        se = jnp.sum(ex, axis=1, keepdims=True)
        probs = ex / se
        val_ref[...] = jnp.max(probs, axis=1, keepdims=True)
        p1gt = probs[:, 1:2] > probs[:, 0:1]
        pred_ref[...] = p1gt.astype(jnp.int32)
        logp = res_g - m - jnp.log(se)
        yv_ = y_ref[...]
        pick = jnp.where(yv_ == 1, logp[:, 1:2], logp[:, 0:1])
        loss_ref[...] = -jnp.sum(pick, keepdims=True).reshape(1, 1) / G


def _head_kernel(loc, h, gid2, y2, scal, s3g, s2g, s3l, s2l, l1T, l2T, lb):
    return pl.pallas_call(
        _head_body,
        grid=(_NBLK,),
        in_specs=[
            pl.BlockSpec((_BN, D), lambda i: (i, 0)),
            pl.BlockSpec((_BN, D), lambda i: (i, 0)),
            pl.BlockSpec((_BN, 1), lambda i: (i, 0)),
            pl.BlockSpec((G, 1), lambda i: (0, 0)),
            pl.BlockSpec(memory_space=pltpu.SMEM),
            pl.BlockSpec((3, 398, 198), lambda i: (0, 0, 0)),
            pl.BlockSpec((2, 198, 99), lambda i: (0, 0, 0)),
            pl.BlockSpec((3, 198, 98), lambda i: (0, 0, 0)),
            pl.BlockSpec((2, 98, 49), lambda i: (0, 0, 0)),
            pl.BlockSpec((99, 2), lambda i: (0, 0)),
            pl.BlockSpec((49, 2), lambda i: (0, 0)),
            pl.BlockSpec((2, 2), lambda i: (0, 0)),
        ],
        out_specs=[
            pl.BlockSpec((G, 1), lambda i: (0, 0)),
            pl.BlockSpec((G, 1), lambda i: (0, 0)),
            pl.BlockSpec((1, 1), lambda i: (0, 0)),
        ],
        out_shape=[
            jax.ShapeDtypeStruct((G, 1), _F32),
            jax.ShapeDtypeStruct((G, 1), jnp.int32),
            jax.ShapeDtypeStruct((1, 1), _F32),
        ],
        scratch_shapes=[
            pltpu.VMEM((G, 2), _F32),
            pltpu.VMEM((G, 1), _F32),
        ],
    )(loc, h, gid2, y2, scal, s3g, s2g, s3l, s2l, l1T, l2T, lb)


def _pool_selectors(L, k, s):
    P = (L - k) // s + 1
    S = np.zeros((k, L, P), np.float32)
    for j in range(P):
        for t in range(k):
            S[t, s * j + t, j] = 1.0
    return S


def kernel(local, edge_index, e_type, graph_ids, y, W_msg, b_msg, W_ih, W_hh,
           b_ih, b_hh, conv1_w, conv1_b, conv2_w, conv2_b, lin1_w, lin1_b,
           lin2_w, lin2_b):
    src = edge_index[0]
    dst = edge_index[1]

    fidx0 = (e_type.astype(jnp.int32) * N + src.astype(jnp.int32)).reshape(NS, NB, BB)
    fidx = jnp.stack([fidx0, fidx0 + NE * N])
    dsti = dst.astype(jnp.int32).reshape(NS, NB, BB)
    zeros = jnp.zeros((RB, DH), _F32)

    wmT = jnp.pad(W_msg.transpose(0, 2, 1),
                  ((0, 0), (0, 0), (0, DP - D)))
    bm = jnp.pad(b_msg, ((0, 0), (0, DP - D))).reshape(NE, 1, DP)
    wihT = W_ih.T
    whhT = W_hh.T
    bih = b_ih.reshape(1, 3 * D)
    bhh = b_hh.reshape(1, 3 * D)

    scal = jnp.stack([conv1_w[0, 0, 0], conv1_w[0, 0, 1], conv1_w[0, 0, 2],
                      conv1_b[0], conv2_w[0, 0, 0], conv2_b[0]])
    s3g = jnp.asarray(_pool_selectors(398, 3, 2))
    s2g = jnp.asarray(_pool_selectors(198, 2, 2))
    s3l = jnp.asarray(_pool_selectors(198, 3, 2))
    s2l = jnp.asarray(_pool_selectors(98, 2, 2))
    l1T = lin1_w.T
    l2T = lin2_w.T
    lb = jnp.stack([lin1_b, lin2_b])

    gid2 = graph_ids.astype(jnp.int32).reshape(N, 1)
    y2 = y.astype(jnp.int32).reshape(G, 1)

    h = local
    ht = _ht_kernel(h, wmT, bm).reshape(NC * NE * N, DH)
    for step in range(STEPS):
        ap = _sc_scatter(ht, fidx, dsti, zeros)
        if step < STEPS - 1:
            h, ht4 = _gru_kernel(h, ap, wihT, whhT, bih, bhh, wmT, bm)
            ht = ht4.reshape(NC * NE * N, DH)
        else:
            h = _gru_last_kernel(h, ap, wihT, whhT, bih, bhh)

    val, pred, loss = _head_kernel(local, h, gid2, y2, scal, s3g, s2g, s3l,
                                   s2l, l1T, l2T, lb)
    return (val.reshape(G), pred.reshape(G), loss.reshape(()))
